# Initial kernel scaffold; baseline (speedup 1.0000x reference)
import jax
import jax.numpy as jnp
from jax.experimental import pallas as pl


def kernel(seq_tokens, x_cov, edge_index, edge_attr, cnn_W, cnn_b, W1, b1, W2, b2, We1, be1, We2, be2):
    raise NotImplementedError("write your pallas kernel here")


# speedup vs baseline: 4.2321x; 4.2321x over previous
import functools

import jax
import jax.numpy as jnp
from jax import lax
from jax.experimental import pallas as pl
from jax.experimental.pallas import tpu as pltpu
from jax.experimental.pallas import tpu_sc as plsc

N = 50000
K = 50
E = 800000
VOCAB = 6
CNN_CH = 32
GCN_H = 64

NPAD = 50048
EPAD = 802816
HALF = 25000
HP = 25088
CH = 128
NC = 2
NS = 16

_mesh = lambda: plsc.VectorSubcoreMesh(core_axis_name="c", subcore_axis_name="s")


def _deg_body(dst_hbm, zeros_hbm, ones_hbm, out_hbm, accum, ones_v, idx_v):
    c = lax.axis_index("c")
    s = lax.axis_index("s")
    rows = NPAD // NS
    pltpu.sync_copy(zeros_hbm, accum.at[pl.ds(s * rows, rows), :])
    pltpu.sync_copy(ones_hbm, ones_v)
    plsc.subcore_barrier()
    epw = EPAD // (NC * NS)
    base = (c * NS + s) * epw

    def chunk(g, carry):
        off = base + g * CH
        pltpu.sync_copy(dst_hbm.at[pl.ds(off, CH)], idx_v)
        pltpu.sync_copy(ones_v, accum.at[idx_v], add=True)
        return carry

    lax.fori_loop(0, epw // CH, chunk, 0)
    plsc.subcore_barrier()
    pltpu.sync_copy(accum.at[pl.ds(s * rows, rows), :],
                    out_hbm.at[c, pl.ds(s * rows, rows), :])


def _deg_parts(dst_p):
    fn = pl.kernel(
        _deg_body,
        out_type=jax.ShapeDtypeStruct((NC, NPAD, 16), jnp.float32),
        mesh=_mesh(),
        compiler_params=pltpu.CompilerParams(use_tc_tiling_on_sc=False),
        scratch_types=[
            pltpu.VMEM_SHARED((NPAD, 16), jnp.float32),
            pltpu.VMEM((CH, 16), jnp.float32),
            pltpu.VMEM((CH,), jnp.int32),
        ],
    )
    zeros = jnp.zeros((NPAD // NS, 16), jnp.float32)
    ones = jnp.ones((CH, 16), jnp.float32)
    return fn(dst_p, zeros, ones)


def _agg_body(src_hbm, dst_hbm, table_hbm, zeros_hbm, out_hbm,
              accum, sidx, didx, rows_v, sem):
    c = lax.axis_index("c")
    s = lax.axis_index("s")
    zrows = HP // NS
    pltpu.sync_copy(zeros_hbm, accum.at[pl.ds(s * zrows, zrows), :])
    plsc.subcore_barrier()
    base_node = c * HALF
    eps = EPAD // NS
    base = s * eps

    def chunk(g, carry):
        off = base + g * CH
        pltpu.sync_copy(src_hbm.at[pl.ds(off, CH)], sidx)
        pltpu.sync_copy(dst_hbm.at[pl.ds(off, CH)], didx)
        for i in range(CH // 16):
            d = didx[pl.ds(i * 16, 16)]
            ld = d - base_node
            ok = (ld >= 0) & (ld < HALF)
            didx[pl.ds(i * 16, 16)] = jnp.where(ok, ld, HALF)
        pltpu.async_copy(table_hbm.at[sidx], rows_v, sem).wait()
        pltpu.sync_copy(rows_v, accum.at[didx], add=True)
        return carry

    lax.fori_loop(0, eps // CH, chunk, 0)
    plsc.subcore_barrier()
    pltpu.sync_copy(accum.at[pl.ds(s * zrows, zrows), :],
                    out_hbm.at[c, pl.ds(s * zrows, zrows), :])


def _aggregate(src_p, dst_p, table):
    fn = pl.kernel(
        _agg_body,
        out_type=jax.ShapeDtypeStruct((NC, HP, GCN_H), jnp.float32),
        mesh=_mesh(),
        compiler_params=pltpu.CompilerParams(use_tc_tiling_on_sc=False),
        scratch_types=[
            pltpu.VMEM_SHARED((HP, GCN_H), jnp.float32),
            pltpu.VMEM((CH,), jnp.int32),
            pltpu.VMEM((CH,), jnp.int32),
            pltpu.VMEM((CH, GCN_H), jnp.float32),
            pltpu.SemaphoreType.DMA,
        ],
    )
    zeros = jnp.zeros((HP // NS, GCN_H), jnp.float32)
    return fn(src_p, dst_p, table, zeros)


def _edge_body(src_hbm, dst_hbm, p_hbm, q_hbm, u_hbm, v_hbm,
               sidx, didx, rows_u, rows_w, sem):
    c = lax.axis_index("c")
    s = lax.axis_index("s")
    epw = EPAD // (NC * NS)
    base = (c * NS + s) * epw

    def chunk(g, carry):
        off = base + g * CH
        pltpu.sync_copy(src_hbm.at[pl.ds(off, CH)], sidx)
        pltpu.sync_copy(dst_hbm.at[pl.ds(off, CH)], didx)
        for i in range(CH // 16):
            d = didx[pl.ds(i * 16, 16)]
            didx[pl.ds(i * 16, 16)] = jnp.minimum(d, N - 1)
        pltpu.async_copy(p_hbm.at[sidx], rows_u, sem).wait()
        pltpu.sync_copy(rows_u, u_hbm.at[pl.ds(off, CH), :])
        pltpu.async_copy(q_hbm.at[didx], rows_w, sem).wait()
        pltpu.sync_copy(rows_w, v_hbm.at[pl.ds(off, CH), :])
        return carry

    lax.fori_loop(0, epw // CH, chunk, 0)


def _edge_gather(src_p, dst_p, p_tab, q_tab):
    fn = pl.kernel(
        _edge_body,
        out_type=(jax.ShapeDtypeStruct((EPAD, GCN_H), jnp.float32),
                  jax.ShapeDtypeStruct((EPAD, GCN_H), jnp.float32)),
        mesh=_mesh(),
        compiler_params=pltpu.CompilerParams(use_tc_tiling_on_sc=False),
        scratch_types=[
            pltpu.VMEM((CH,), jnp.int32),
            pltpu.VMEM((CH,), jnp.int32),
            pltpu.VMEM((CH, GCN_H), jnp.float32),
            pltpu.VMEM((CH, GCN_H), jnp.float32),
            pltpu.SemaphoreType.DMA,
        ],
    )
    return fn(src_p, dst_p, p_tab, q_tab)


BN_CNN = 200


def _cnn_body(tok_ref, xcov_ref, wf_ref, cb_ref, w1a_ref, w1r_ref, out_ref):
    tok = tok_ref[...]
    pad = jnp.full((BN_CNN, 1), 100, dtype=jnp.int32)
    t0 = jnp.concatenate([pad, tok[:, : K - 1]], axis=1)
    t1 = tok + 6
    t2 = jnp.concatenate([tok[:, 1:] + 12, pad], axis=1)
    iota18 = lax.broadcasted_iota(jnp.int32, (BN_CNN, K, 18), 2)
    z = ((t0[:, :, None] == iota18).astype(jnp.float32)
         + (t1[:, :, None] == iota18).astype(jnp.float32)
         + (t2[:, :, None] == iota18).astype(jnp.float32))
    zf = z.reshape(BN_CNN * K, 18)
    conv = lax.dot_general(zf, wf_ref[...], (((1,), (0,)), ((), ())),
                           preferred_element_type=jnp.float32)
    r = jnp.maximum(conv + cb_ref[...], 0.0)
    h = jnp.sum(r.reshape(BN_CNN, K, CNN_CH), axis=1) * (1.0 / K)
    xw = lax.dot_general(h, w1a_ref[...], (((1,), (0,)), ((), ())),
                         preferred_element_type=jnp.float32)
    out_ref[...] = xw + xcov_ref[...] * w1r_ref[...]


def _cnn_xw(seq_tokens, x_cov, cnn_W, cnn_b, W1):
    wf = jnp.concatenate([cnn_W[:, :, 0].T, cnn_W[:, :, 1].T, cnn_W[:, :, 2].T],
                         axis=0)
    return pl.pallas_call(
        _cnn_body,
        grid=(N // BN_CNN,),
        in_specs=[
            pl.BlockSpec((BN_CNN, K), lambda i: (i, 0)),
            pl.BlockSpec((BN_CNN, 1), lambda i: (i, 0)),
            pl.BlockSpec((3 * VOCAB, CNN_CH), lambda i: (0, 0)),
            pl.BlockSpec((1, CNN_CH), lambda i: (0, 0)),
            pl.BlockSpec((CNN_CH, GCN_H), lambda i: (0, 0)),
            pl.BlockSpec((1, GCN_H), lambda i: (0, 0)),
        ],
        out_specs=pl.BlockSpec((BN_CNN, GCN_H), lambda i: (i, 0)),
        out_shape=jax.ShapeDtypeStruct((N, GCN_H), jnp.float32),
    )(seq_tokens, x_cov, wf, cnn_b.reshape(1, CNN_CH), W1[:CNN_CH],
      W1[CNN_CH:CNN_CH + 1])


BN_S = 1000


def _scale_body(dp_ref, xw_ref, xwn_ref, dinv_ref):
    dp = dp_ref[...]
    deg = dp[0, :, 0:1] + dp[1, :, 0:1] + 1.0
    dv = lax.rsqrt(deg)
    dinv_ref[...] = dv
    xwn_ref[...] = xw_ref[...] * dv


def _scale(deg_parts, xw):
    return pl.pallas_call(
        _scale_body,
        grid=(N // BN_S,),
        in_specs=[
            pl.BlockSpec((NC, BN_S, 16), lambda i: (0, i, 0)),
            pl.BlockSpec((BN_S, GCN_H), lambda i: (i, 0)),
        ],
        out_specs=[
            pl.BlockSpec((BN_S, GCN_H), lambda i: (i, 0)),
            pl.BlockSpec((BN_S, 1), lambda i: (i, 0)),
        ],
        out_shape=[
            jax.ShapeDtypeStruct((N, GCN_H), jnp.float32),
            jax.ShapeDtypeStruct((N, 1), jnp.float32),
        ],
    )(deg_parts, xw)


def _layer_body(s_ref, xn_ref, dv_ref, w_ref, b_ref, out_ref):
    dv = dv_ref[...]
    h = jnp.maximum(dv * (s_ref[0] + xn_ref[...]) + b_ref[...], 0.0)
    hw = lax.dot_general(h, w_ref[...], (((1,), (0,)), ((), ())),
                         preferred_element_type=jnp.float32)
    out_ref[...] = hw * dv


def _layer(S, xwn, dinv, w2, b):
    return pl.pallas_call(
        _layer_body,
        grid=(N // BN_S,),
        in_specs=[
            pl.BlockSpec((1, BN_S, GCN_H), lambda i: (i // 25, i % 25, 0)),
            pl.BlockSpec((BN_S, GCN_H), lambda i: (i, 0)),
            pl.BlockSpec((BN_S, 1), lambda i: (i, 0)),
            pl.BlockSpec((GCN_H, GCN_H), lambda i: (0, 0)),
            pl.BlockSpec((1, GCN_H), lambda i: (0, 0)),
        ],
        out_specs=pl.BlockSpec((BN_S, GCN_H), lambda i: (i, 0)),
        out_shape=jax.ShapeDtypeStruct((N, GCN_H), jnp.float32),
    )(S, xwn, dinv, w2, b.reshape(1, GCN_H))


def _proj_body(s_ref, xn_ref, dv_ref, a_ref, bb_ref, b2_ref, be1_ref,
               p_ref, q_ref):
    dv = dv_ref[...]
    h = jnp.maximum(dv * (s_ref[0] + xn_ref[...]) + b2_ref[...], 0.0)
    p_ref[...] = lax.dot_general(h, a_ref[...], (((1,), (0,)), ((), ())),
                                 preferred_element_type=jnp.float32)
    q_ref[...] = lax.dot_general(h, bb_ref[...], (((1,), (0,)), ((), ())),
                                 preferred_element_type=jnp.float32) + be1_ref[...]


def _proj(S2, hwn, dinv, We1, b2, be1):
    return pl.pallas_call(
        _proj_body,
        grid=(N // BN_S,),
        in_specs=[
            pl.BlockSpec((1, BN_S, GCN_H), lambda i: (i // 25, i % 25, 0)),
            pl.BlockSpec((BN_S, GCN_H), lambda i: (i, 0)),
            pl.BlockSpec((BN_S, 1), lambda i: (i, 0)),
            pl.BlockSpec((GCN_H, GCN_H), lambda i: (0, 0)),
            pl.BlockSpec((GCN_H, GCN_H), lambda i: (0, 0)),
            pl.BlockSpec((1, GCN_H), lambda i: (0, 0)),
            pl.BlockSpec((1, GCN_H), lambda i: (0, 0)),
        ],
        out_specs=[
            pl.BlockSpec((BN_S, GCN_H), lambda i: (i, 0)),
            pl.BlockSpec((BN_S, GCN_H), lambda i: (i, 0)),
        ],
        out_shape=[
            jax.ShapeDtypeStruct((N, GCN_H), jnp.float32),
            jax.ShapeDtypeStruct((N, GCN_H), jnp.float32),
        ],
    )(S2, hwn, dinv, We1[:GCN_H], We1[GCN_H:2 * GCN_H],
      b2.reshape(1, GCN_H), be1.reshape(1, GCN_H))


BE = 2048


def _emlp_body(u_ref, v_ref, ea_ref, c2_ref, w2_ref, be2_ref, out_ref):
    pre = u_ref[...] + v_ref[...] + lax.dot_general(
        ea_ref[...], c2_ref[...], (((1,), (0,)), ((), ())),
        preferred_element_type=jnp.float32)
    z = jnp.maximum(pre, 0.0)
    out_ref[...] = lax.dot_general(z, w2_ref[...], (((1,), (0,)), ((), ())),
                                   preferred_element_type=jnp.float32) + be2_ref[...]


def _edge_mlp(U, V, eap, We1, We2, be2):
    c2 = jnp.concatenate([We1[2 * GCN_H:], jnp.zeros((3, GCN_H), jnp.float32)],
                         axis=0)
    return pl.pallas_call(
        _emlp_body,
        grid=(EPAD // BE,),
        in_specs=[
            pl.BlockSpec((BE, GCN_H), lambda i: (i, 0)),
            pl.BlockSpec((BE, GCN_H), lambda i: (i, 0)),
            pl.BlockSpec((BE, 8), lambda i: (i, 0)),
            pl.BlockSpec((8, GCN_H), lambda i: (0, 0)),
            pl.BlockSpec((GCN_H, 1), lambda i: (0, 0)),
            pl.BlockSpec((1, 1), lambda i: (0, 0)),
        ],
        out_specs=pl.BlockSpec((BE, 1), lambda i: (i, 0)),
        out_shape=jax.ShapeDtypeStruct((EPAD, 1), jnp.float32),
    )(U, V, eap, c2, We2, be2.reshape(1, 1))


def kernel(seq_tokens, x_cov, edge_index, edge_attr, cnn_W, cnn_b,
           W1, b1, W2, b2, We1, be1, We2, be2):
    src = edge_index[0].astype(jnp.int32)
    dst = edge_index[1].astype(jnp.int32)
    src_p = jnp.concatenate([src, jnp.zeros((EPAD - E,), jnp.int32)])
    dst_p = jnp.concatenate([dst, jnp.full((EPAD - E,), N, jnp.int32)])

    deg_parts = _deg_parts(dst_p)
    xw = _cnn_xw(seq_tokens.astype(jnp.int32), x_cov, cnn_W, cnn_b, W1)
    xwn, dinv = _scale(deg_parts, xw)

    S1 = _aggregate(src_p, dst_p, xwn)
    hwn = _layer(S1, xwn, dinv, W2, b1)
    S2 = _aggregate(src_p, dst_p, hwn)
    P, Q = _proj(S2, hwn, dinv, We1, b2, be1)

    U, V = _edge_gather(src_p, dst_p, P, Q)
    eap = jnp.pad(edge_attr, ((0, EPAD - E), (0, 3)))
    out = _edge_mlp(U, V, eap, We1, We2, be2)
    return out[:E, 0]


# speedup vs baseline: 5.0704x; 1.1981x over previous
import functools

import jax
import jax.numpy as jnp
from jax import lax
from jax.experimental import pallas as pl
from jax.experimental.pallas import tpu as pltpu
from jax.experimental.pallas import tpu_sc as plsc

N = 50000
K = 50
E = 800000
VOCAB = 6
CNN_CH = 32
GCN_H = 64

NPAD = 50048
EPAD = 802816
HALF = 25000
HP = 25088
CH = 128
NC = 2
NS = 16
NBUF = 4
ABUF = 2

_mesh = lambda: plsc.VectorSubcoreMesh(core_axis_name="c", subcore_axis_name="s")
_sc_params = lambda: pltpu.CompilerParams(use_tc_tiling_on_sc=False)


def _deg_body(dst_hbm, zeros_hbm, ones_hbm, out_hbm, accum, ones_v, idx_v,
              *sems):
    isems, ssems = sems[:NBUF], sems[NBUF:]
    c = lax.axis_index("c")
    s = lax.axis_index("s")
    rows = NPAD // NS
    pltpu.sync_copy(zeros_hbm, accum.at[pl.ds(s * rows, rows), :])
    pltpu.sync_copy(ones_hbm, ones_v)
    plsc.subcore_barrier()
    epw = EPAD // (NC * NS)
    base = (c * NS + s) * epw
    nch = epw // CH

    def issue_idx(g, b):
        pltpu.async_copy(dst_hbm.at[pl.ds(base + g * CH, CH)], idx_v.at[b],
                         isems[b])

    def wait_idx(b):
        pltpu.make_async_copy(dst_hbm.at[pl.ds(0, CH)], idx_v.at[b],
                              isems[b]).wait()

    def wait_scat(b):
        pltpu.make_async_copy(ones_v, accum.at[idx_v.at[b]], ssems[b]).wait()

    for b in range(NBUF - 1):
        issue_idx(b, b)

    def step(t, carry):
        for b in range(NBUF):
            g = t * NBUF + b
            prev = (b + NBUF - 1) % NBUF
            wait_idx(b)
            pltpu.async_copy(ones_v, accum.at[idx_v.at[b]], ssems[b],
                             add=True)

            @pl.when(g >= 1)
            def _():
                wait_scat(prev)

            @pl.when(g + NBUF - 1 < nch)
            def _():
                issue_idx(g + NBUF - 1, prev)
        return carry

    lax.fori_loop(0, nch // NBUF, step, 0)
    wait_scat((nch - 1) % NBUF)
    plsc.subcore_barrier()
    pltpu.sync_copy(accum.at[pl.ds(s * rows, rows), :],
                    out_hbm.at[c, pl.ds(s * rows, rows), :])


def _deg_parts(dst_p):
    fn = pl.kernel(
        _deg_body,
        out_type=jax.ShapeDtypeStruct((NC, NPAD, 16), jnp.float32),
        mesh=_mesh(),
        compiler_params=_sc_params(),
        scratch_types=[
            pltpu.VMEM_SHARED((NPAD, 16), jnp.float32),
            pltpu.VMEM((CH, 16), jnp.float32),
            pltpu.VMEM((NBUF, CH), jnp.int32),
        ] + [pltpu.SemaphoreType.DMA] * (2 * NBUF),
    )
    zeros = jnp.zeros((NPAD // NS, 16), jnp.float32)
    ones = jnp.ones((CH, 16), jnp.float32)
    return fn(dst_p, zeros, ones)


def _agg_body(src_hbm, dst_hbm, table_hbm, zeros_hbm, out_hbm,
              accum, sidx, didx, didx2, rows_v, *sems):
    isems = sems[:ABUF]
    gsems = sems[ABUF:2 * ABUF]
    ssems = sems[2 * ABUF:]
    c = lax.axis_index("c")
    s = lax.axis_index("s")
    zrows = HP // NS
    pltpu.sync_copy(zeros_hbm, accum.at[pl.ds(s * zrows, zrows), :])
    plsc.subcore_barrier()
    base_node = c * HALF
    eps = EPAD // NS
    base = s * eps
    nch = eps // CH

    def issue_idx(g, b):
        off = base + g * CH
        pltpu.async_copy(src_hbm.at[pl.ds(off, CH)], sidx.at[b], isems[b])
        pltpu.async_copy(dst_hbm.at[pl.ds(off, CH)], didx.at[b], isems[b])

    def wait_idx(b):
        pltpu.make_async_copy(src_hbm.at[pl.ds(0, CH)], sidx.at[b],
                              isems[b]).wait()
        pltpu.make_async_copy(dst_hbm.at[pl.ds(0, CH)], didx.at[b],
                              isems[b]).wait()

    def wait_gather(b):
        pltpu.make_async_copy(table_hbm.at[sidx.at[b]], rows_v.at[b],
                              gsems[b]).wait()

    def issue_scat(b):
        pltpu.async_copy(rows_v.at[b], accum.at[didx2.at[b]], ssems[b],
                         add=True)

    def wait_scat(b):
        pltpu.make_async_copy(rows_v.at[b], accum.at[didx2.at[b]],
                              ssems[b]).wait()

    for b in range(ABUF - 1):
        issue_idx(b, b)

    def step(t, carry):
        for b in range(ABUF):
            g = t * ABUF + b
            prev = (b + ABUF - 1) % ABUF

            @pl.when(g >= ABUF)
            def _():
                wait_scat(b)

            wait_idx(b)
            for i in range(CH // 16):
                d = didx[b, pl.ds(i * 16, 16)]
                ld = d - base_node
                ok = (ld >= 0) & (ld < HALF)
                didx2[b, pl.ds(i * 16, 16)] = jnp.where(ok, ld, HALF)
            pltpu.async_copy(table_hbm.at[sidx.at[b]], rows_v.at[b], gsems[b])

            @pl.when(g >= 1)
            def _():
                wait_gather(prev)
                issue_scat(prev)

            @pl.when(g + ABUF - 1 < nch)
            def _():
                issue_idx(g + ABUF - 1, prev)
        return carry

    lax.fori_loop(0, nch // ABUF, step, 0)
    last = (nch - 1) % ABUF
    wait_gather(last)
    issue_scat(last)
    for b in range(ABUF):
        wait_scat(b)
    plsc.subcore_barrier()
    pltpu.sync_copy(accum.at[pl.ds(s * zrows, zrows), :],
                    out_hbm.at[c, pl.ds(s * zrows, zrows), :])


def _aggregate(src_p, dst_p, table):
    fn = pl.kernel(
        _agg_body,
        out_type=jax.ShapeDtypeStruct((NC, HP, GCN_H), jnp.float32),
        mesh=_mesh(),
        compiler_params=_sc_params(),
        scratch_types=[
            pltpu.VMEM_SHARED((HP, GCN_H), jnp.float32),
            pltpu.VMEM((ABUF, CH), jnp.int32),
            pltpu.VMEM((ABUF, CH), jnp.int32),
            pltpu.VMEM((ABUF, CH), jnp.int32),
            pltpu.VMEM((ABUF, CH, GCN_H), jnp.float32),
        ] + [pltpu.SemaphoreType.DMA] * (3 * ABUF),
    )
    zeros = jnp.zeros((HP // NS, GCN_H), jnp.float32)
    return fn(src_p, dst_p, table, zeros)


def _edge_body(src_hbm, dst_hbm, p_hbm, q_hbm, u_hbm, v_hbm,
               sidx, didx, rows_u, rows_w, *sems):
    isems = sems[:NBUF]
    gusems = sems[NBUF:2 * NBUF]
    gvsems = sems[2 * NBUF:3 * NBUF]
    wusems = sems[3 * NBUF:4 * NBUF]
    wvsems = sems[4 * NBUF:]
    c = lax.axis_index("c")
    s = lax.axis_index("s")
    epw = EPAD // (NC * NS)
    base = (c * NS + s) * epw
    nch = epw // CH

    def issue_idx(g, b):
        off = base + g * CH
        pltpu.async_copy(src_hbm.at[pl.ds(off, CH)], sidx.at[b], isems[b])
        pltpu.async_copy(dst_hbm.at[pl.ds(off, CH)], didx.at[b], isems[b])

    def wait_idx(b):
        pltpu.make_async_copy(src_hbm.at[pl.ds(0, CH)], sidx.at[b],
                              isems[b]).wait()
        pltpu.make_async_copy(dst_hbm.at[pl.ds(0, CH)], didx.at[b],
                              isems[b]).wait()

    def wait_gathers(b):
        pltpu.make_async_copy(p_hbm.at[sidx.at[b]], rows_u.at[b],
                              gusems[b]).wait()
        pltpu.make_async_copy(q_hbm.at[didx.at[b]], rows_w.at[b],
                              gvsems[b]).wait()

    def issue_writes(g, b):
        off = base + g * CH
        pltpu.async_copy(rows_u.at[b], u_hbm.at[pl.ds(off, CH), :], wusems[b])
        pltpu.async_copy(rows_w.at[b], v_hbm.at[pl.ds(off, CH), :], wvsems[b])

    def wait_writes(b):
        pltpu.make_async_copy(rows_u.at[b], u_hbm.at[pl.ds(0, CH), :],
                              wusems[b]).wait()
        pltpu.make_async_copy(rows_w.at[b], v_hbm.at[pl.ds(0, CH), :],
                              wvsems[b]).wait()

    for b in range(NBUF - 1):
        issue_idx(b, b)

    def step(t, carry):
        for b in range(NBUF):
            g = t * NBUF + b
            prev = (b + NBUF - 1) % NBUF

            @pl.when(g >= NBUF)
            def _():
                wait_writes(b)

            wait_idx(b)
            for i in range(CH // 16):
                d = didx[b, pl.ds(i * 16, 16)]
                didx[b, pl.ds(i * 16, 16)] = jnp.minimum(d, N - 1)
            pltpu.async_copy(p_hbm.at[sidx.at[b]], rows_u.at[b], gusems[b])
            pltpu.async_copy(q_hbm.at[didx.at[b]], rows_w.at[b], gvsems[b])

            @pl.when(g >= 1)
            def _():
                wait_gathers(prev)
                issue_writes(g - 1, prev)

            @pl.when(g + NBUF - 1 < nch)
            def _():
                issue_idx(g + NBUF - 1, prev)
        return carry

    lax.fori_loop(0, nch // NBUF, step, 0)
    last = (nch - 1) % NBUF
    wait_gathers(last)
    issue_writes(nch - 1, last)
    for b in range(NBUF):
        wait_writes(b)


def _edge_gather(src_p, dst_p, p_tab, q_tab):
    fn = pl.kernel(
        _edge_body,
        out_type=(jax.ShapeDtypeStruct((EPAD, GCN_H), jnp.float32),
                  jax.ShapeDtypeStruct((EPAD, GCN_H), jnp.float32)),
        mesh=_mesh(),
        compiler_params=_sc_params(),
        scratch_types=[
            pltpu.VMEM((NBUF, CH), jnp.int32),
            pltpu.VMEM((NBUF, CH), jnp.int32),
            pltpu.VMEM((NBUF, CH, GCN_H), jnp.float32),
            pltpu.VMEM((NBUF, CH, GCN_H), jnp.float32),
        ] + [pltpu.SemaphoreType.DMA] * (5 * NBUF),
    )
    return fn(src_p, dst_p, p_tab, q_tab)


BN_CNN = 200


def _cnn_body(tok_ref, xcov_ref, wf_ref, cb_ref, w1a_ref, w1r_ref, out_ref):
    tok = tok_ref[...]
    pad = jnp.full((BN_CNN, 1), 100, dtype=jnp.int32)
    t0 = jnp.concatenate([pad, tok[:, : K - 1]], axis=1)
    t1 = tok + 6
    t2 = jnp.concatenate([tok[:, 1:] + 12, pad], axis=1)
    iota18 = lax.broadcasted_iota(jnp.int32, (BN_CNN, K, 18), 2)
    z = ((t0[:, :, None] == iota18).astype(jnp.float32)
         + (t1[:, :, None] == iota18).astype(jnp.float32)
         + (t2[:, :, None] == iota18).astype(jnp.float32))
    zf = z.reshape(BN_CNN * K, 18)
    conv = lax.dot_general(zf, wf_ref[...], (((1,), (0,)), ((), ())),
                           preferred_element_type=jnp.float32)
    r = jnp.maximum(conv + cb_ref[...], 0.0)
    h = jnp.sum(r.reshape(BN_CNN, K, CNN_CH), axis=1) * (1.0 / K)
    xw = lax.dot_general(h, w1a_ref[...], (((1,), (0,)), ((), ())),
                         preferred_element_type=jnp.float32)
    out_ref[...] = xw + xcov_ref[...] * w1r_ref[...]


def _cnn_xw(seq_tokens, x_cov, cnn_W, cnn_b, W1):
    wf = jnp.concatenate([cnn_W[:, :, 0].T, cnn_W[:, :, 1].T, cnn_W[:, :, 2].T],
                         axis=0)
    return pl.pallas_call(
        _cnn_body,
        grid=(N // BN_CNN,),
        in_specs=[
            pl.BlockSpec((BN_CNN, K), lambda i: (i, 0)),
            pl.BlockSpec((BN_CNN, 1), lambda i: (i, 0)),
            pl.BlockSpec((3 * VOCAB, CNN_CH), lambda i: (0, 0)),
            pl.BlockSpec((1, CNN_CH), lambda i: (0, 0)),
            pl.BlockSpec((CNN_CH, GCN_H), lambda i: (0, 0)),
            pl.BlockSpec((1, GCN_H), lambda i: (0, 0)),
        ],
        out_specs=pl.BlockSpec((BN_CNN, GCN_H), lambda i: (i, 0)),
        out_shape=jax.ShapeDtypeStruct((N, GCN_H), jnp.float32),
    )(seq_tokens, x_cov, wf, cnn_b.reshape(1, CNN_CH), W1[:CNN_CH],
      W1[CNN_CH:CNN_CH + 1])


BN_S = 1000


def _scale_body(dp_ref, xw_ref, xwn_ref, dinv_ref):
    dp = dp_ref[...]
    deg = dp[0, :, 0:1] + dp[1, :, 0:1] + 1.0
    dv = lax.rsqrt(deg)
    dinv_ref[...] = dv
    xwn_ref[...] = xw_ref[...] * dv


def _scale(deg_parts, xw):
    return pl.pallas_call(
        _scale_body,
        grid=(N // BN_S,),
        in_specs=[
            pl.BlockSpec((NC, BN_S, 16), lambda i: (0, i, 0)),
            pl.BlockSpec((BN_S, GCN_H), lambda i: (i, 0)),
        ],
        out_specs=[
            pl.BlockSpec((BN_S, GCN_H), lambda i: (i, 0)),
            pl.BlockSpec((BN_S, 1), lambda i: (i, 0)),
        ],
        out_shape=[
            jax.ShapeDtypeStruct((N, GCN_H), jnp.float32),
            jax.ShapeDtypeStruct((N, 1), jnp.float32),
        ],
    )(deg_parts, xw)


def _layer_body(s_ref, xn_ref, dv_ref, w_ref, b_ref, out_ref):
    dv = dv_ref[...]
    h = jnp.maximum(dv * (s_ref[0] + xn_ref[...]) + b_ref[...], 0.0)
    hw = lax.dot_general(h, w_ref[...], (((1,), (0,)), ((), ())),
                         preferred_element_type=jnp.float32)
    out_ref[...] = hw * dv


def _layer(S, xwn, dinv, w2, b):
    return pl.pallas_call(
        _layer_body,
        grid=(N // BN_S,),
        in_specs=[
            pl.BlockSpec((1, BN_S, GCN_H), lambda i: (i // 25, i % 25, 0)),
            pl.BlockSpec((BN_S, GCN_H), lambda i: (i, 0)),
            pl.BlockSpec((BN_S, 1), lambda i: (i, 0)),
            pl.BlockSpec((GCN_H, GCN_H), lambda i: (0, 0)),
            pl.BlockSpec((1, GCN_H), lambda i: (0, 0)),
        ],
        out_specs=pl.BlockSpec((BN_S, GCN_H), lambda i: (i, 0)),
        out_shape=jax.ShapeDtypeStruct((N, GCN_H), jnp.float32),
    )(S, xwn, dinv, w2, b.reshape(1, GCN_H))


def _proj_body(s_ref, xn_ref, dv_ref, a_ref, bb_ref, b2_ref, be1_ref,
               p_ref, q_ref):
    dv = dv_ref[...]
    h = jnp.maximum(dv * (s_ref[0] + xn_ref[...]) + b2_ref[...], 0.0)
    p_ref[...] = lax.dot_general(h, a_ref[...], (((1,), (0,)), ((), ())),
                                 preferred_element_type=jnp.float32)
    q_ref[...] = lax.dot_general(h, bb_ref[...], (((1,), (0,)), ((), ())),
                                 preferred_element_type=jnp.float32) + be1_ref[...]


def _proj(S2, hwn, dinv, We1, b2, be1):
    return pl.pallas_call(
        _proj_body,
        grid=(N // BN_S,),
        in_specs=[
            pl.BlockSpec((1, BN_S, GCN_H), lambda i: (i // 25, i % 25, 0)),
            pl.BlockSpec((BN_S, GCN_H), lambda i: (i, 0)),
            pl.BlockSpec((BN_S, 1), lambda i: (i, 0)),
            pl.BlockSpec((GCN_H, GCN_H), lambda i: (0, 0)),
            pl.BlockSpec((GCN_H, GCN_H), lambda i: (0, 0)),
            pl.BlockSpec((1, GCN_H), lambda i: (0, 0)),
            pl.BlockSpec((1, GCN_H), lambda i: (0, 0)),
        ],
        out_specs=[
            pl.BlockSpec((BN_S, GCN_H), lambda i: (i, 0)),
            pl.BlockSpec((BN_S, GCN_H), lambda i: (i, 0)),
        ],
        out_shape=[
            jax.ShapeDtypeStruct((N, GCN_H), jnp.float32),
            jax.ShapeDtypeStruct((N, GCN_H), jnp.float32),
        ],
    )(S2, hwn, dinv, We1[:GCN_H], We1[GCN_H:2 * GCN_H],
      b2.reshape(1, GCN_H), be1.reshape(1, GCN_H))


BE = 2048


def _emlp_body(u_ref, v_ref, ea_ref, c2_ref, w2_ref, be2_ref, out_ref):
    pre = u_ref[...] + v_ref[...] + lax.dot_general(
        ea_ref[...], c2_ref[...], (((1,), (0,)), ((), ())),
        preferred_element_type=jnp.float32)
    z = jnp.maximum(pre, 0.0)
    out_ref[...] = lax.dot_general(z, w2_ref[...], (((1,), (0,)), ((), ())),
                                   preferred_element_type=jnp.float32) + be2_ref[...]


def _edge_mlp(U, V, eap, We1, We2, be2):
    c2 = jnp.concatenate([We1[2 * GCN_H:], jnp.zeros((3, GCN_H), jnp.float32)],
                         axis=0)
    return pl.pallas_call(
        _emlp_body,
        grid=(EPAD // BE,),
        in_specs=[
            pl.BlockSpec((BE, GCN_H), lambda i: (i, 0)),
            pl.BlockSpec((BE, GCN_H), lambda i: (i, 0)),
            pl.BlockSpec((BE, 8), lambda i: (i, 0)),
            pl.BlockSpec((8, GCN_H), lambda i: (0, 0)),
            pl.BlockSpec((GCN_H, 1), lambda i: (0, 0)),
            pl.BlockSpec((1, 1), lambda i: (0, 0)),
        ],
        out_specs=pl.BlockSpec((BE, 1), lambda i: (i, 0)),
        out_shape=jax.ShapeDtypeStruct((EPAD, 1), jnp.float32),
    )(U, V, eap, c2, We2, be2.reshape(1, 1))


def kernel(seq_tokens, x_cov, edge_index, edge_attr, cnn_W, cnn_b,
           W1, b1, W2, b2, We1, be1, We2, be2):
    src = edge_index[0].astype(jnp.int32)
    dst = edge_index[1].astype(jnp.int32)
    src_p = jnp.concatenate([src, jnp.zeros((EPAD - E,), jnp.int32)])
    dst_p = jnp.concatenate([dst, jnp.full((EPAD - E,), N, jnp.int32)])

    deg_parts = _deg_parts(dst_p)
    xw = _cnn_xw(seq_tokens.astype(jnp.int32), x_cov, cnn_W, cnn_b, W1)
    xwn, dinv = _scale(deg_parts, xw)

    S1 = _aggregate(src_p, dst_p, xwn)
    hwn = _layer(S1, xwn, dinv, W2, b1)
    S2 = _aggregate(src_p, dst_p, hwn)
    P, Q = _proj(S2, hwn, dinv, We1, b2, be1)

    U, V = _edge_gather(src_p, dst_p, P, Q)
    eap = jnp.pad(edge_attr, ((0, EPAD - E), (0, 3)))
    out = _edge_mlp(U, V, eap, We1, We2, be2)
    return out[:E, 0]


# speedup vs baseline: 6.2877x; 1.2401x over previous
import functools

import jax
import jax.numpy as jnp
from jax import lax
from jax.experimental import pallas as pl
from jax.experimental.pallas import tpu as pltpu
from jax.experimental.pallas import tpu_sc as plsc

N = 50000
K = 50
E = 800000
VOCAB = 6
CNN_CH = 32
GCN_H = 64

NPAD = 50048
EPAD = 802816
HALF = 25000
HP = 25088
CH = 128
NC = 2
NS = 16
NBUF = 4
ABUF = 2
NPAD3 = 50176
NPAIR = 25
GC = 224
TQ = 2401

_mesh = lambda: plsc.VectorSubcoreMesh(core_axis_name="c", subcore_axis_name="s")
_sc_params = lambda: pltpu.CompilerParams(use_tc_tiling_on_sc=False)


def _deg_body(dst_hbm, zeros_hbm, ones_hbm, out_hbm, accum, ones_v, idx_v,
              *sems):
    isems, ssems = sems[:NBUF], sems[NBUF:]
    c = lax.axis_index("c")
    s = lax.axis_index("s")
    rows = NPAD // NS
    pltpu.sync_copy(zeros_hbm, accum.at[pl.ds(s * rows, rows), :])
    pltpu.sync_copy(ones_hbm, ones_v)
    plsc.subcore_barrier()
    epw = EPAD // (NC * NS)
    base = (c * NS + s) * epw
    nch = epw // CH

    def issue_idx(g, b):
        pltpu.async_copy(dst_hbm.at[pl.ds(base + g * CH, CH)], idx_v.at[b],
                         isems[b])

    def wait_idx(b):
        pltpu.make_async_copy(dst_hbm.at[pl.ds(0, CH)], idx_v.at[b],
                              isems[b]).wait()

    def wait_scat(b):
        pltpu.make_async_copy(ones_v, accum.at[idx_v.at[b]], ssems[b]).wait()

    for b in range(NBUF - 1):
        issue_idx(b, b)

    def step(t, carry):
        for b in range(NBUF):
            g = t * NBUF + b
            prev = (b + NBUF - 1) % NBUF
            wait_idx(b)
            pltpu.async_copy(ones_v, accum.at[idx_v.at[b]], ssems[b],
                             add=True)

            @pl.when(g >= 1)
            def _():
                wait_scat(prev)

            @pl.when(g + NBUF - 1 < nch)
            def _():
                issue_idx(g + NBUF - 1, prev)
        return carry

    lax.fori_loop(0, nch // NBUF, step, 0)
    wait_scat((nch - 1) % NBUF)
    plsc.subcore_barrier()
    pltpu.sync_copy(accum.at[pl.ds(s * rows, rows), :],
                    out_hbm.at[c, pl.ds(s * rows, rows), :])


def _deg_parts(dst_p):
    fn = pl.kernel(
        _deg_body,
        out_type=jax.ShapeDtypeStruct((NC, NPAD, 16), jnp.float32),
        mesh=_mesh(),
        compiler_params=_sc_params(),
        scratch_types=[
            pltpu.VMEM_SHARED((NPAD, 16), jnp.float32),
            pltpu.VMEM((CH, 16), jnp.float32),
            pltpu.VMEM((NBUF, CH), jnp.int32),
        ] + [pltpu.SemaphoreType.DMA] * (2 * NBUF),
    )
    zeros = jnp.zeros((NPAD // NS, 16), jnp.float32)
    ones = jnp.ones((CH, 16), jnp.float32)
    return fn(dst_p, zeros, ones)


def _agg_body(src_hbm, dst_hbm, table_hbm, zeros_hbm, out_hbm,
              accum, sidx, didx, didx2, rows_v, *sems):
    isems = sems[:ABUF]
    gsems = sems[ABUF:2 * ABUF]
    ssems = sems[2 * ABUF:]
    c = lax.axis_index("c")
    s = lax.axis_index("s")
    zrows = HP // NS
    pltpu.sync_copy(zeros_hbm, accum.at[pl.ds(s * zrows, zrows), :])
    plsc.subcore_barrier()
    base_node = c * HALF
    eps = EPAD // NS
    base = s * eps
    nch = eps // CH

    def issue_idx(g, b):
        off = base + g * CH
        pltpu.async_copy(src_hbm.at[pl.ds(off, CH)], sidx.at[b], isems[b])
        pltpu.async_copy(dst_hbm.at[pl.ds(off, CH)], didx.at[b], isems[b])

    def wait_idx(b):
        pltpu.make_async_copy(src_hbm.at[pl.ds(0, CH)], sidx.at[b],
                              isems[b]).wait()
        pltpu.make_async_copy(dst_hbm.at[pl.ds(0, CH)], didx.at[b],
                              isems[b]).wait()

    def wait_gather(b):
        pltpu.make_async_copy(table_hbm.at[sidx.at[b]], rows_v.at[b],
                              gsems[b]).wait()

    def issue_scat(b):
        pltpu.async_copy(rows_v.at[b], accum.at[didx2.at[b]], ssems[b],
                         add=True)

    def wait_scat(b):
        pltpu.make_async_copy(rows_v.at[b], accum.at[didx2.at[b]],
                              ssems[b]).wait()

    for b in range(ABUF - 1):
        issue_idx(b, b)

    def step(t, carry):
        for b in range(ABUF):
            g = t * ABUF + b
            prev = (b + ABUF - 1) % ABUF

            @pl.when(g >= ABUF)
            def _():
                wait_scat(b)

            wait_idx(b)
            for i in range(CH // 16):
                d = didx[b, pl.ds(i * 16, 16)]
                ld = d - base_node
                ok = (ld >= 0) & (ld < HALF)
                didx2[b, pl.ds(i * 16, 16)] = jnp.where(ok, ld, HALF)
            pltpu.async_copy(table_hbm.at[sidx.at[b]], rows_v.at[b], gsems[b])

            @pl.when(g >= 1)
            def _():
                wait_gather(prev)
                issue_scat(prev)

            @pl.when(g + ABUF - 1 < nch)
            def _():
                issue_idx(g + ABUF - 1, prev)
        return carry

    lax.fori_loop(0, nch // ABUF, step, 0)
    last = (nch - 1) % ABUF
    wait_gather(last)
    issue_scat(last)
    for b in range(ABUF):
        wait_scat(b)
    plsc.subcore_barrier()
    pltpu.sync_copy(accum.at[pl.ds(s * zrows, zrows), :],
                    out_hbm.at[c, pl.ds(s * zrows, zrows), :])


def _aggregate(src_p, dst_p, table):
    fn = pl.kernel(
        _agg_body,
        out_type=jax.ShapeDtypeStruct((NC, HP, GCN_H), jnp.float32),
        mesh=_mesh(),
        compiler_params=_sc_params(),
        scratch_types=[
            pltpu.VMEM_SHARED((HP, GCN_H), jnp.float32),
            pltpu.VMEM((ABUF, CH), jnp.int32),
            pltpu.VMEM((ABUF, CH), jnp.int32),
            pltpu.VMEM((ABUF, CH), jnp.int32),
            pltpu.VMEM((ABUF, CH, GCN_H), jnp.float32),
        ] + [pltpu.SemaphoreType.DMA] * (3 * ABUF),
    )
    zeros = jnp.zeros((HP // NS, GCN_H), jnp.float32)
    return fn(src_p, dst_p, table, zeros)


def _edge_body(src_hbm, dst_hbm, p_hbm, q_hbm, u_hbm, v_hbm,
               sidx, didx, rows_u, rows_w, *sems):
    isems = sems[:NBUF]
    gusems = sems[NBUF:2 * NBUF]
    gvsems = sems[2 * NBUF:3 * NBUF]
    wusems = sems[3 * NBUF:4 * NBUF]
    wvsems = sems[4 * NBUF:]
    c = lax.axis_index("c")
    s = lax.axis_index("s")
    epw = EPAD // (NC * NS)
    base = (c * NS + s) * epw
    nch = epw // CH

    def issue_idx(g, b):
        off = base + g * CH
        pltpu.async_copy(src_hbm.at[pl.ds(off, CH)], sidx.at[b], isems[b])
        pltpu.async_copy(dst_hbm.at[pl.ds(off, CH)], didx.at[b], isems[b])

    def wait_idx(b):
        pltpu.make_async_copy(src_hbm.at[pl.ds(0, CH)], sidx.at[b],
                              isems[b]).wait()
        pltpu.make_async_copy(dst_hbm.at[pl.ds(0, CH)], didx.at[b],
                              isems[b]).wait()

    def wait_gathers(b):
        pltpu.make_async_copy(p_hbm.at[sidx.at[b]], rows_u.at[b],
                              gusems[b]).wait()
        pltpu.make_async_copy(q_hbm.at[didx.at[b]], rows_w.at[b],
                              gvsems[b]).wait()

    def issue_writes(g, b):
        off = base + g * CH
        pltpu.async_copy(rows_u.at[b], u_hbm.at[pl.ds(off, CH), :], wusems[b])
        pltpu.async_copy(rows_w.at[b], v_hbm.at[pl.ds(off, CH), :], wvsems[b])

    def wait_writes(b):
        pltpu.make_async_copy(rows_u.at[b], u_hbm.at[pl.ds(0, CH), :],
                              wusems[b]).wait()
        pltpu.make_async_copy(rows_w.at[b], v_hbm.at[pl.ds(0, CH), :],
                              wvsems[b]).wait()

    for b in range(NBUF - 1):
        issue_idx(b, b)

    def step(t, carry):
        for b in range(NBUF):
            g = t * NBUF + b
            prev = (b + NBUF - 1) % NBUF

            @pl.when(g >= NBUF)
            def _():
                wait_writes(b)

            wait_idx(b)
            for i in range(CH // 16):
                d = didx[b, pl.ds(i * 16, 16)]
                didx[b, pl.ds(i * 16, 16)] = jnp.minimum(d, N - 1)
            pltpu.async_copy(p_hbm.at[sidx.at[b]], rows_u.at[b], gusems[b])
            pltpu.async_copy(q_hbm.at[didx.at[b]], rows_w.at[b], gvsems[b])

            @pl.when(g >= 1)
            def _():
                wait_gathers(prev)
                issue_writes(g - 1, prev)

            @pl.when(g + NBUF - 1 < nch)
            def _():
                issue_idx(g + NBUF - 1, prev)
        return carry

    lax.fori_loop(0, nch // NBUF, step, 0)
    last = (nch - 1) % NBUF
    wait_gathers(last)
    issue_writes(nch - 1, last)
    for b in range(NBUF):
        wait_writes(b)


def _edge_gather(src_p, dst_p, p_tab, q_tab):
    fn = pl.kernel(
        _edge_body,
        out_type=(jax.ShapeDtypeStruct((EPAD, GCN_H), jnp.float32),
                  jax.ShapeDtypeStruct((EPAD, GCN_H), jnp.float32)),
        mesh=_mesh(),
        compiler_params=_sc_params(),
        scratch_types=[
            pltpu.VMEM((NBUF, CH), jnp.int32),
            pltpu.VMEM((NBUF, CH), jnp.int32),
            pltpu.VMEM((NBUF, CH, GCN_H), jnp.float32),
            pltpu.VMEM((NBUF, CH, GCN_H), jnp.float32),
        ] + [pltpu.SemaphoreType.DMA] * (5 * NBUF),
    )
    return fn(src_p, dst_p, p_tab, q_tab)


BN_S = 1000


def _qid_body(tok_ref, wsel_ref, out_ref):
    tokf = tok_ref[...].astype(jnp.float32)
    pad = jnp.full((BN_S, 1), 6.0, dtype=jnp.float32)
    tokp = jnp.concatenate([pad, tokf, pad], axis=1)
    qidf = lax.dot_general(tokp, wsel_ref[...], (((1,), (0,)), ((), ())),
                           preferred_element_type=jnp.float32)
    out_ref[...] = (qidf + 0.5).astype(jnp.int32)


def _quad_ids(seq_tokens):
    import numpy as _np
    wsel = _np.zeros((K + 2, NPAIR), _np.float32)
    for j in range(NPAIR):
        wsel[2 * j, j] = 343.0
        wsel[2 * j + 1, j] = 49.0
        wsel[2 * j + 2, j] = 7.0
        wsel[2 * j + 3, j] = 1.0
    return pl.pallas_call(
        _qid_body,
        grid=(N // BN_S,),
        in_specs=[
            pl.BlockSpec((BN_S, K), lambda i: (i, 0)),
            pl.BlockSpec((K + 2, NPAIR), lambda i: (0, 0)),
        ],
        out_specs=pl.BlockSpec((BN_S, NPAIR), lambda i: (i, 0)),
        out_shape=jax.ShapeDtypeStruct((NPAD3, NPAIR), jnp.int32),
    )(seq_tokens, jnp.asarray(wsel))


def _cnnsc_body(qid_hbm, t4_hbm, h_hbm, t4v, qch, hch):
    c = lax.axis_index("c")
    s = lax.axis_index("s")
    pltpu.sync_copy(t4_hbm, t4v)
    npw = NPAD3 // (NC * NS)
    base = (c * NS + s) * npw
    iota16 = lax.iota(jnp.int32, 16)
    nchk = npw // GC

    def chunk(gg, carry):
        noff = base + gg * GC
        pltpu.sync_copy(qid_hbm.at[pl.ds(noff * NPAIR, GC * NPAIR)], qch)

        def group(gi, carry2):
            q0 = gi * (16 * NPAIR) + iota16 * NPAIR

            def pair(j, acc_t):
                accl = list(acc_t)
                q = plsc.load_gather(qch, [q0 + j])
                q = jnp.minimum(jnp.maximum(q, 0), TQ - 1)
                bidx = q * CNN_CH
                for w2 in range(CNN_CH):
                    g = plsc.load_gather(t4v, [bidx + w2])
                    accl[w2] = accl[w2] + g
                return tuple(accl)

            acc = lax.fori_loop(0, NPAIR, pair,
                                tuple([jnp.zeros((16,), jnp.float32)] * CNN_CH))
            for w2 in range(CNN_CH):
                hch[w2, pl.ds(gi * 16, 16)] = acc[w2]
            return carry2

        lax.fori_loop(0, GC // 16, group, 0)
        pltpu.sync_copy(hch, h_hbm.at[(noff // GC)])
        return carry

    lax.fori_loop(0, nchk, chunk, 0)


def _cnn_lookup(qid, t4f):
    fn = pl.kernel(
        _cnnsc_body,
        out_type=jax.ShapeDtypeStruct((NPAD3 // GC, CNN_CH, GC), jnp.float32),
        mesh=_mesh(),
        compiler_params=pltpu.CompilerParams(use_tc_tiling_on_sc=False,
                                             needs_layout_passes=False),
        scratch_types=[
            pltpu.VMEM((TQ * CNN_CH,), jnp.float32),
            pltpu.VMEM((GC * NPAIR,), jnp.int32),
            pltpu.VMEM((CNN_CH, GC), jnp.float32),
        ],
    )
    h3 = fn(qid.reshape(NPAD3 * NPAIR), t4f)
    return jnp.transpose(h3, (0, 2, 1)).reshape(NPAD3, CNN_CH)


def _t4_table(cnn_W, cnn_b):
    z1 = jnp.zeros((1, CNN_CH), jnp.float32)
    a7 = jnp.concatenate([cnn_W[:, :, 0].T, z1], axis=0)
    b7 = jnp.concatenate([cnn_W[:, :, 1].T, z1], axis=0)
    c7 = jnp.concatenate([cnn_W[:, :, 2].T, z1], axis=0)
    qs = jnp.arange(TQ)
    q0, q1 = qs // 343, (qs // 49) % 7
    q2, q3 = (qs // 7) % 7, qs % 7
    t4 = (jax.nn.relu(a7[q0] + b7[q1] + c7[q2] + cnn_b)
          + jax.nn.relu(a7[q1] + b7[q2] + c7[q3] + cnn_b)) * (1.0 / K)
    return t4.reshape(-1)


def _scale_body(dp_ref, h_ref, xc_ref, w1a_ref, w1r_ref, xwn_ref, dinv_ref):
    dp = dp_ref[...]
    deg = dp[0, :, 0:1] + dp[1, :, 0:1] + 1.0
    dv = lax.rsqrt(deg)
    dinv_ref[...] = dv
    xw = lax.dot_general(h_ref[...], w1a_ref[...], (((1,), (0,)), ((), ())),
                         preferred_element_type=jnp.float32)
    xwn_ref[...] = (xw + xc_ref[...] * w1r_ref[...]) * dv


def _scale(deg_parts, h, x_cov, W1):
    return pl.pallas_call(
        _scale_body,
        grid=(N // BN_S,),
        in_specs=[
            pl.BlockSpec((NC, BN_S, 16), lambda i: (0, i, 0)),
            pl.BlockSpec((BN_S, CNN_CH), lambda i: (i, 0)),
            pl.BlockSpec((BN_S, 1), lambda i: (i, 0)),
            pl.BlockSpec((CNN_CH, GCN_H), lambda i: (0, 0)),
            pl.BlockSpec((1, GCN_H), lambda i: (0, 0)),
        ],
        out_specs=[
            pl.BlockSpec((BN_S, GCN_H), lambda i: (i, 0)),
            pl.BlockSpec((BN_S, 1), lambda i: (i, 0)),
        ],
        out_shape=[
            jax.ShapeDtypeStruct((N, GCN_H), jnp.float32),
            jax.ShapeDtypeStruct((N, 1), jnp.float32),
        ],
    )(deg_parts, h, x_cov, W1[:CNN_CH], W1[CNN_CH:CNN_CH + 1])


def _layer_body(s_ref, xn_ref, dv_ref, w_ref, b_ref, out_ref):
    dv = dv_ref[...]
    h = jnp.maximum(dv * (s_ref[0] + xn_ref[...]) + b_ref[...], 0.0)
    hw = lax.dot_general(h, w_ref[...], (((1,), (0,)), ((), ())),
                         preferred_element_type=jnp.float32)
    out_ref[...] = hw * dv


def _layer(S, xwn, dinv, w2, b):
    return pl.pallas_call(
        _layer_body,
        grid=(N // BN_S,),
        in_specs=[
            pl.BlockSpec((1, BN_S, GCN_H), lambda i: (i // 25, i % 25, 0)),
            pl.BlockSpec((BN_S, GCN_H), lambda i: (i, 0)),
            pl.BlockSpec((BN_S, 1), lambda i: (i, 0)),
            pl.BlockSpec((GCN_H, GCN_H), lambda i: (0, 0)),
            pl.BlockSpec((1, GCN_H), lambda i: (0, 0)),
        ],
        out_specs=pl.BlockSpec((BN_S, GCN_H), lambda i: (i, 0)),
        out_shape=jax.ShapeDtypeStruct((N, GCN_H), jnp.float32),
    )(S, xwn, dinv, w2, b.reshape(1, GCN_H))


def _proj_body(s_ref, xn_ref, dv_ref, a_ref, bb_ref, b2_ref, be1_ref,
               p_ref, q_ref):
    dv = dv_ref[...]
    h = jnp.maximum(dv * (s_ref[0] + xn_ref[...]) + b2_ref[...], 0.0)
    p_ref[...] = lax.dot_general(h, a_ref[...], (((1,), (0,)), ((), ())),
                                 preferred_element_type=jnp.float32)
    q_ref[...] = lax.dot_general(h, bb_ref[...], (((1,), (0,)), ((), ())),
                                 preferred_element_type=jnp.float32) + be1_ref[...]


def _proj(S2, hwn, dinv, We1, b2, be1):
    return pl.pallas_call(
        _proj_body,
        grid=(N // BN_S,),
        in_specs=[
            pl.BlockSpec((1, BN_S, GCN_H), lambda i: (i // 25, i % 25, 0)),
            pl.BlockSpec((BN_S, GCN_H), lambda i: (i, 0)),
            pl.BlockSpec((BN_S, 1), lambda i: (i, 0)),
            pl.BlockSpec((GCN_H, GCN_H), lambda i: (0, 0)),
            pl.BlockSpec((GCN_H, GCN_H), lambda i: (0, 0)),
            pl.BlockSpec((1, GCN_H), lambda i: (0, 0)),
            pl.BlockSpec((1, GCN_H), lambda i: (0, 0)),
        ],
        out_specs=[
            pl.BlockSpec((BN_S, GCN_H), lambda i: (i, 0)),
            pl.BlockSpec((BN_S, GCN_H), lambda i: (i, 0)),
        ],
        out_shape=[
            jax.ShapeDtypeStruct((N, GCN_H), jnp.float32),
            jax.ShapeDtypeStruct((N, GCN_H), jnp.float32),
        ],
    )(S2, hwn, dinv, We1[:GCN_H], We1[GCN_H:2 * GCN_H],
      b2.reshape(1, GCN_H), be1.reshape(1, GCN_H))


BE = 1000


def _emlp_body(u_ref, v_ref, ea_ref, c2_ref, w2_ref, be2_ref, out_ref):
    pre = u_ref[...] + v_ref[...] + lax.dot_general(
        ea_ref[...], c2_ref[...], (((1,), (0,)), ((), ())),
        preferred_element_type=jnp.float32)
    z = jnp.maximum(pre, 0.0)
    out_ref[...] = lax.dot_general(z, w2_ref[...], (((1,), (0,)), ((), ())),
                                   preferred_element_type=jnp.float32) + be2_ref[...]


def _edge_mlp(U, V, edge_attr, We1, We2, be2):
    return pl.pallas_call(
        _emlp_body,
        grid=(E // BE,),
        in_specs=[
            pl.BlockSpec((BE, GCN_H), lambda i: (i, 0)),
            pl.BlockSpec((BE, GCN_H), lambda i: (i, 0)),
            pl.BlockSpec((BE, 5), lambda i: (i, 0)),
            pl.BlockSpec((5, GCN_H), lambda i: (0, 0)),
            pl.BlockSpec((GCN_H, 1), lambda i: (0, 0)),
            pl.BlockSpec((1, 1), lambda i: (0, 0)),
        ],
        out_specs=pl.BlockSpec((BE, 1), lambda i: (i, 0)),
        out_shape=jax.ShapeDtypeStruct((E, 1), jnp.float32),
    )(U, V, edge_attr, We1[2 * GCN_H:], We2, be2.reshape(1, 1))


def kernel(seq_tokens, x_cov, edge_index, edge_attr, cnn_W, cnn_b,
           W1, b1, W2, b2, We1, be1, We2, be2):
    src = edge_index[0].astype(jnp.int32)
    dst = edge_index[1].astype(jnp.int32)
    src_p = jnp.concatenate([src, jnp.zeros((EPAD - E,), jnp.int32)])
    dst_p = jnp.concatenate([dst, jnp.full((EPAD - E,), N, jnp.int32)])

    deg_parts = _deg_parts(dst_p)
    qid = _quad_ids(seq_tokens.astype(jnp.int32))
    h = _cnn_lookup(qid, _t4_table(cnn_W, cnn_b))
    xwn, dinv = _scale(deg_parts, h[:N], x_cov, W1)

    S1 = _aggregate(src_p, dst_p, xwn)
    hwn = _layer(S1, xwn, dinv, W2, b1)
    S2 = _aggregate(src_p, dst_p, hwn)
    P, Q = _proj(S2, hwn, dinv, We1, b2, be1)

    U, V = _edge_gather(src_p, dst_p, P, Q)
    out = _edge_mlp(U, V, edge_attr, We1, We2, be2)
    return out[:, 0]
